# baseline (speedup 1.0000x reference)
import functools

import jax
import jax.numpy as jnp
from jax import lax
from jax.experimental import pallas as pl
from jax.experimental.pallas import tpu as pltpu
from jax.experimental.pallas import tpu_sc as plsc

N = 10000
NPAD = 10240
D = 128
DW = 64
NG = 64
NCORES = 2
NSUB = 16
NTILES = NCORES * NSUB
BLK = 128
HB = 64
NBLK = 80
EPT = NBLK * BLK
EPAD = NTILES * EPT
E0 = 320000
RPT = NPAD // NSUB

_f32 = jnp.float32
_i32 = jnp.int32

_PERM = tuple(2 * (c * 16 + i) + q
              for c in range(4) for q in range(2) for i in range(16))


_ROWB = 1024
_GRID = NPAD // _ROWB


def _prep_body(x_ref, w_ref, asw_ref, adw_ref, hp_ref, asv_ref, adv_ref):
    h = jnp.dot(x_ref[...], w_ref[...], preferred_element_type=_f32)
    hp_ref[...] = h.astype(jnp.bfloat16)
    asv_ref[...] = jnp.sum(h * asw_ref[...][None, :], axis=1)
    adv_ref[...] = jnp.sum(h * adw_ref[...][None, :], axis=1)


def _prep(x_pad, W, asw, adw):
    return pl.pallas_call(
        _prep_body,
        grid=(_GRID,),
        in_specs=[
            pl.BlockSpec((_ROWB, D), lambda i: (i, 0)),
            pl.BlockSpec((D, D), lambda i: (0, 0)),
            pl.BlockSpec((D,), lambda i: (0,)),
            pl.BlockSpec((D,), lambda i: (0,)),
        ],
        out_specs=[
            pl.BlockSpec((_ROWB, D), lambda i: (i, 0)),
            pl.BlockSpec((_ROWB,), lambda i: (i,)),
            pl.BlockSpec((_ROWB,), lambda i: (i,)),
        ],
        out_shape=[
            jax.ShapeDtypeStruct((NPAD, D), jnp.bfloat16),
            jax.ShapeDtypeStruct((NPAD,), _f32),
            jax.ShapeDtypeStruct((NPAD,), _f32),
        ],
    )(x_pad, W, asw, adw)


def _merge_body(acc_ref, den_ref, b_ref, w_ref, asw_ref, adw_ref,
                hp_ref, asv_ref, adv_ref):
    den = den_ref[0] + den_ref[1] + _f32(1e-16)
    out = (acc_ref[0] + acc_ref[1]) / den[:, None] + b_ref[...][None, :]
    hin = jnp.maximum(out, _f32(0.0))
    h = jnp.dot(hin, w_ref[...], preferred_element_type=_f32)
    hp_ref[...] = h.astype(jnp.bfloat16)
    asv_ref[...] = jnp.sum(h * asw_ref[...][None, :], axis=1)
    adv_ref[...] = jnp.sum(h * adw_ref[...][None, :], axis=1)


def _merge(accp, denp, b, W, asw, adw):
    return pl.pallas_call(
        _merge_body,
        grid=(_GRID,),
        in_specs=[
            pl.BlockSpec((2, _ROWB, D), lambda i: (0, i, 0)),
            pl.BlockSpec((2, _ROWB), lambda i: (0, i)),
            pl.BlockSpec((D,), lambda i: (0,)),
            pl.BlockSpec((D, D), lambda i: (0, 0)),
            pl.BlockSpec((D,), lambda i: (0,)),
            pl.BlockSpec((D,), lambda i: (0,)),
        ],
        out_specs=[
            pl.BlockSpec((_ROWB, D), lambda i: (i, 0)),
            pl.BlockSpec((_ROWB,), lambda i: (i,)),
            pl.BlockSpec((_ROWB,), lambda i: (i,)),
        ],
        out_shape=[
            jax.ShapeDtypeStruct((NPAD, D), jnp.bfloat16),
            jax.ShapeDtypeStruct((NPAD,), _f32),
            jax.ShapeDtypeStruct((NPAD,), _f32),
        ],
    )(accp, denp, b, W, asw, adw)


def _final_body(acc_ref, den_ref, b_ref, batch_ref, lw_ref, lb_ref,
                y_ref, sums_ref, cnt_ref):
    i = pl.program_id(0)

    @pl.when(i == 0)
    def _():
        sums_ref[...] = jnp.zeros_like(sums_ref)
        cnt_ref[...] = jnp.zeros_like(cnt_ref)

    den = den_ref[0] + den_ref[1] + _f32(1e-16)
    out = (acc_ref[0] + acc_ref[1]) / den[:, None] + b_ref[...][None, :]
    oh = (lax.broadcasted_iota(_i32, (NG, _ROWB), 0)
          == batch_ref[...][None, :]).astype(_f32)
    sums_ref[...] += jnp.dot(oh, out, preferred_element_type=_f32)
    cnt_ref[...] += jnp.sum(oh, axis=1)

    @pl.when(i == pl.num_programs(0) - 1)
    def _():
        pooled = sums_ref[...] / jnp.maximum(cnt_ref[...], _f32(1.0))[:, None]
        y_ref[...] = (jnp.dot(pooled, lw_ref[...], preferred_element_type=_f32)
                      + lb_ref[...][None, :])


def _final(accp, denp, b, batch_pad, lin_W, lin_b):
    return pl.pallas_call(
        _final_body,
        grid=(_GRID,),
        in_specs=[
            pl.BlockSpec((2, _ROWB, D), lambda i: (0, i, 0)),
            pl.BlockSpec((2, _ROWB), lambda i: (0, i)),
            pl.BlockSpec((D,), lambda i: (0,)),
            pl.BlockSpec((_ROWB,), lambda i: (i,)),
            pl.BlockSpec((D, D), lambda i: (0, 0)),
            pl.BlockSpec((D,), lambda i: (0,)),
        ],
        out_specs=pl.BlockSpec((NG, D), lambda i: (0, 0)),
        out_shape=jax.ShapeDtypeStruct((NG, D), _f32),
        scratch_shapes=[
            pltpu.VMEM((NG, D), _f32),
            pltpu.VMEM((NG,), _f32),
        ],
    )(accp, denp, b, batch_pad, lin_W, lin_b)


def _edge_body(hp_hbm, asv_hbm, adv_hbm, srci_hbm, dsti_hbm,
               accp_hbm, denp_hbm,
               sidx_v, didx_v, asb_v, adb_v, e_v, rows_v, stg_v,
               acc_sh, den_sh, asv_sh, adv_sh,
               sem_i, sem_a, sem_g, sem_sc, sem_dn):
    cid = lax.axis_index("c")
    sid = lax.axis_index("s")
    wid = cid * NSUB + sid
    zv = jnp.zeros((16,), _f32)
    m16 = jnp.int32(-65536)

    def _zrow(i, _):
        for k in range(8):
            stg_v[0, i, pl.ds(k * 16, 16)] = zv
        return 0
    lax.fori_loop(0, HB, _zrow, 0)
    for k in range(4):
        e_v[0, 0, pl.ds(k * 16, 16)] = zv
    for z in range(RPT // HB):
        pltpu.sync_copy(stg_v.at[0],
                        acc_sh.at[pl.ds(sid * RPT + z * HB, HB)])
        pltpu.sync_copy(e_v.at[0, 0],
                        den_sh.at[pl.ds(sid * RPT + z * HB, HB)])
    @pl.when(sid == 0)
    def _():
        pltpu.sync_copy(asv_hbm, asv_sh)
        pltpu.sync_copy(adv_hbm, adv_sh)
    plsc.subcore_barrier()

    def _prefetch_idx(jn, s6):
        pltpu.async_copy(srci_hbm.at[wid, jn], sidx_v.at[s6], sem_i)
        pltpu.async_copy(dsti_hbm.at[wid, jn], didx_v.at[s6], sem_i)

    def _wait_idx(jn, s6):
        pltpu.make_async_copy(srci_hbm.at[wid, jn], sidx_v.at[s6],
                              sem_i).wait()
        pltpu.make_async_copy(dsti_hbm.at[wid, jn], didx_v.at[s6],
                              sem_i).wait()

    def _issue_gathers(s6, t3):
        pltpu.async_copy(asv_sh.at[sidx_v.at[s6]], asb_v.at[t3], sem_a)
        pltpu.async_copy(adv_sh.at[didx_v.at[s6, 0]], adb_v.at[t3, 0], sem_a)
        pltpu.async_copy(adv_sh.at[didx_v.at[s6, 1]], adb_v.at[t3, 1], sem_a)
        pltpu.async_copy(hp_hbm.at[sidx_v.at[s6]], rows_v.at[t3], sem_g)

    for g in range(4):
        _prefetch_idx(g, g)
    _wait_idx(0, 0)
    _issue_gathers(0, 0)
    _wait_idx(1, 1)
    _issue_gathers(1, 1)

    def _block(j, _):
        c2 = lax.rem(j, 2)
        p2 = lax.rem(j + 1, 2)
        c3 = lax.rem(j, 3)
        n3 = lax.rem(j + 2, 3)
        c6 = lax.rem(j, 6)
        n6 = lax.rem(j + 2, 6)
        f6 = lax.rem(j + 4, 6)
        p6 = lax.rem(j + 5, 6)

        @pl.when(j >= 1)
        def _():
            for h in range(2):
                pltpu.make_async_copy(e_v.at[p2, h],
                                      den_sh.at[didx_v.at[p6, h]],
                                      sem_dn).wait()

        @pl.when(j + 2 < NBLK)
        def _():
            _wait_idx(j + 2, n6)
            _issue_gathers(n6, n3)

        @pl.when(j + 4 < NBLK)
        def _():
            _prefetch_idx(j + 4, f6)

        pltpu.make_async_copy(asv_sh.at[sidx_v.at[c6]], asb_v.at[c3],
                              sem_a).wait()
        for h in range(2):
            pltpu.make_async_copy(adv_sh.at[didx_v.at[c6, h]],
                                  adb_v.at[c3, h], sem_a).wait()
        for h in range(2):
            for k in range(4):
                sl = pl.ds(k * 16, 16)
                a = asb_v[c3, pl.ds(h * HB + k * 16, 16)] + adb_v[c3, h, sl]
                a = jnp.where(a >= 0, a, a * _f32(0.2))
                e_v[c2, h, sl] = jnp.exp(a)
            pltpu.async_copy(e_v.at[c2, h], den_sh.at[didx_v.at[c6, h]],
                             sem_dn, add=True)

        pltpu.make_async_copy(hp_hbm.at[sidx_v.at[c6]], rows_v.at[c3],
                              sem_g).wait()

        for h in range(2):
            @pl.when(j >= 1)
            def _():
                pltpu.make_async_copy(stg_v.at[h],
                                      acc_sh.at[didx_v.at[p6, h]],
                                      sem_sc).wait()

            @plsc.parallel_loop(0, HB, unroll=4)
            def _scale(i):
                s = plsc.load_gather(e_v.at[c2, h],
                                     [jnp.full((16,), i, _i32)])
                for cc in range(4):
                    v = plsc.bitcast(
                        rows_v[c3, h * HB + i, pl.ds(cc * 32, 32)], _i32)
                    flo = plsc.bitcast(v << 16, _f32)
                    fhi = plsc.bitcast(v & m16, _f32)
                    stg_v[h, i, pl.ds(cc * 32, 16)] = flo * s
                    stg_v[h, i, pl.ds(cc * 32 + 16, 16)] = fhi * s

            pltpu.async_copy(stg_v.at[h], acc_sh.at[didx_v.at[c6, h]],
                             sem_sc, add=True)
        return 0
    lax.fori_loop(0, NBLK, _block, 0)

    l2 = (NBLK - 1) % 2
    l6 = (NBLK - 1) % 6
    for h in range(2):
        pltpu.make_async_copy(e_v.at[l2, h], den_sh.at[didx_v.at[l6, h]],
                              sem_dn).wait()
        pltpu.make_async_copy(stg_v.at[h], acc_sh.at[didx_v.at[l6, h]],
                              sem_sc).wait()

    plsc.subcore_barrier()

    for z in range(RPT // BLK):
        r0 = sid * RPT + z * BLK
        pltpu.sync_copy(acc_sh.at[pl.ds(r0, BLK)],
                        accp_hbm.at[cid, pl.ds(r0, BLK)])

    @pl.when(sid == 0)
    def _():
        pltpu.sync_copy(den_sh, denp_hbm.at[cid])


_edge = functools.partial(
    pl.kernel,
    out_type=[
        jax.ShapeDtypeStruct((NCORES, NPAD, D), _f32),
        jax.ShapeDtypeStruct((NCORES, NPAD), _f32),
    ],
    mesh=plsc.VectorSubcoreMesh(core_axis_name="c", subcore_axis_name="s"),
    compiler_params=pltpu.CompilerParams(needs_layout_passes=False,
                                         use_tc_tiling_on_sc=False),
    scratch_types=[
        pltpu.VMEM((6, BLK), _i32),
        pltpu.VMEM((6, 2, HB), _i32),
        pltpu.VMEM((3, BLK), _f32),
        pltpu.VMEM((3, 2, HB), _f32),
        pltpu.VMEM((2, 2, HB), _f32),
        pltpu.VMEM((3, BLK, D), jnp.bfloat16),
        pltpu.VMEM((2, HB, D), _f32),
        pltpu.VMEM_SHARED((NPAD, D), _f32),
        pltpu.VMEM_SHARED((NPAD,), _f32),
        pltpu.VMEM_SHARED((NPAD,), _f32),
        pltpu.VMEM_SHARED((NPAD,), _f32),
        pltpu.SemaphoreType.DMA,
        pltpu.SemaphoreType.DMA,
        pltpu.SemaphoreType.DMA,
        pltpu.SemaphoreType.DMA,
        pltpu.SemaphoreType.DMA,
    ],
)(_edge_body)


def kernel(x, edge_index, edge_attr, batch,
           W1, b1, as1, ad1, W2, b2, as2, ad2, W3, b3, as3, ad3,
           lin_W, lin_b):
    perm = jnp.array(_PERM, dtype=_i32)
    src = edge_index[0].astype(_i32)
    dst = edge_index[1].astype(_i32)
    pad_idx = (jnp.arange(EPAD - E0, dtype=_i32) % (NPAD - N)) + N
    srcp = jnp.concatenate([src, pad_idx]).reshape(NTILES, NBLK, BLK)
    dstp = jnp.concatenate([dst, pad_idx]).reshape(NTILES, NBLK, 2, HB)
    x_pad = jnp.pad(x, ((0, NPAD - N), (0, 0)))
    batch_pad = jnp.pad(batch.astype(_i32), (0, NPAD - N),
                        constant_values=NG)

    hp, asv, adv = _prep(x_pad, W1, as1, ad1)

    W_st = jnp.stack([W2[perm, :], W3[perm, :], W3[perm, :]])
    as_st = jnp.stack([as2, as3, as3])
    ad_st = jnp.stack([ad2, ad3, ad3])
    b_st = jnp.stack([b1[perm], b2[perm], b2[perm]])
    acc0 = jnp.zeros((NCORES, NPAD, D), _f32)
    den0 = jnp.zeros((NCORES, NPAD), _f32)

    def _layer(carry, ws):
        hc, asvc, advc, _, _ = carry
        W, asw, adw, b = ws
        accp, denp = _edge(hc, asvc, advc, srcp, dstp)
        hn, asvn, advn = _merge(accp, denp, b, W, asw, adw)
        return (hn, asvn, advn, accp, denp), None

    (_, _, _, accp, denp), _ = lax.scan(
        _layer, (hp, asv, adv, acc0, den0), (W_st, as_st, ad_st, b_st))
    return _final(accp, denp, b3[perm], batch_pad, lin_W[perm, :], lin_b)
